# baseline (speedup 1.0000x reference)
import functools

import jax
import jax.numpy as jnp
from jax import lax
from jax.experimental import pallas as pl
from jax.experimental.pallas import tpu as pltpu
from jax.experimental.pallas import tpu_sc as plsc

USER_COUNT = 25000
ITEM_COUNT = 25000
N_NODES = USER_COUNT + ITEM_COUNT
N_EDGES = 800000
EMB = 64
HALF = 32
BATCH = 4096
N_LAYERS = 3

NC = 2
NT = 16

EPT = N_EDGES // NT
CHUNK = 128
CPT = 392
EPTP = CPT * CHUNK
E_PAD = NT * EPTP
BLK = 28
NBUF = 4
NSB = 2
DEPTH = 3
NBLK = CPT // BLK
EBLK = BLK * CHUNK

NP = 50048
RPT = NP // NT
OUT_ROWS = 2 * BATCH
ORPT = OUT_ROWS // NT
OMB = ORPT // CHUNK


def _lgcn_body(ego0, rows2, cols1, vals1, idx1, zeros_in,
               out_h, e1, e2, e3,
               acc, rows_v, cols_v, vals_v, gb0, gb1, gb2, gb3, sb0, sb1,
               idx_v, mrows_v,
               gs0, gs1, gs2, gs3, ss0, ss1, sem):
    cid = lax.axis_index("c")
    tid = lax.axis_index("s")
    iota16 = lax.iota(jnp.int32, 16)

    srcs = [ego0, e1, e2, e3]
    gbufs = (gb0, gb1, gb2, gb3)
    gsems = (gs0, gs1, gs2, gs3)
    sbufs = (sb0, sb1)
    ssems = (ss0, ss1)

    for layer in range(N_LAYERS):
        src = srcs[layer]
        dst = srcs[layer + 1]
        pltpu.sync_copy(zeros_in, acc.at[pl.ds(tid * RPT, RPT)])
        plsc.subcore_barrier()

        def blk_body(b, _, src=src):
            cb = tid * CPT + b * BLK
            eb = cb * CHUNK
            pltpu.sync_copy(rows2.at[pl.ds(cb, BLK)], rows_v)
            pltpu.sync_copy(cols1.at[pl.ds(cid * E_PAD + eb, EBLK)], cols_v)
            pltpu.sync_copy(vals1.at[pl.ds(eb, EBLK)], vals_v)

            def fire(kq, p):
                pltpu.async_copy(
                    src.at[cols_v.at[pl.ds(kq * CHUNK, CHUNK)]],
                    gbufs[p], gsems[p])

            def scale(gb, sb, k):
                def grp_body(g, _):
                    vv = vals_v[pl.ds(k * CHUNK + g * 16, 16)]
                    for j in range(16):
                        bv = vv.at[jnp.full((16,), j, jnp.int32)].get(
                            mode="promise_in_bounds")
                        r = g * 16 + j
                        v = gb[r, pl.ds(0, 16)]
                        a = plsc.bitcast(lax.shift_left(v, 16), jnp.float32)
                        b2 = plsc.bitcast(v & jnp.int32(-65536), jnp.float32)
                        sb[r, pl.ds(0, 16)] = a * bv
                        sb[r, pl.ds(16, 16)] = b2 * bv
                    return 0

                lax.fori_loop(0, CHUNK // 16, grp_body, 0)

            for p in range(DEPTH):
                fire(p, p)

            def quad_body(kk, _):
                for p in range(NBUF):
                    k = kk * NBUF + p
                    s = p % NSB
                    pltpu.make_async_copy(
                        ego0.at[pl.ds(0, CHUNK)], gbufs[p], gsems[p]
                    ).wait()
                    if p >= NSB:
                        pltpu.make_async_copy(
                            zeros_in.at[pl.ds(0, CHUNK)], sbufs[s], ssems[s]
                        ).wait()
                    else:
                        @pl.when(kk > 0)
                        def _(s=s):
                            pltpu.make_async_copy(
                                zeros_in.at[pl.ds(0, CHUNK)], sbufs[s],
                                ssems[s]).wait()

                    @pl.when(k + DEPTH < BLK)
                    def _(k=k, p=p):
                        fire(k + DEPTH, (p + DEPTH) % NBUF)

                    scale(gbufs[p], sbufs[s], k)
                    pltpu.async_copy(sbufs[s], acc.at[rows_v.at[k]],
                                     ssems[s], add=True)
                return 0

            lax.fori_loop(0, BLK // NBUF, quad_body, 0)
            for k in range(BLK - NSB, BLK):
                pltpu.make_async_copy(
                    zeros_in.at[pl.ds(0, CHUNK)], sbufs[k % NSB],
                    ssems[k % NSB]).wait()
            return 0

        lax.fori_loop(0, NBLK, blk_body, 0)
        plsc.subcore_barrier()
        for i in range(RPT // CHUNK + 1):
            n = min(CHUNK, RPT - i * CHUNK)
            off = tid * RPT + i * CHUNK
            pltpu.sync_copy(acc.at[pl.ds(off, n)], sb0.at[pl.ds(0, n)])

            def wb_body(r, _):
                ai = plsc.bitcast(sb0[r, pl.ds(0, 16)], jnp.int32)
                bi = plsc.bitcast(sb0[r, pl.ds(16, 16)], jnp.int32)
                lo = lax.shift_right_logical(ai + jnp.int32(0x8000), 16)
                hi = (bi + jnp.int32(0x8000)) & jnp.int32(-65536)
                gb0[r, pl.ds(0, 16)] = lo | hi
                return 0

            lax.fori_loop(0, n, wb_body, 0)
            pltpu.sync_copy(gb0.at[pl.ds(0, n)],
                            dst.at[pl.ds(cid * NP + off, n)])
        plsc.subcore_barrier()

    pltpu.sync_copy(idx1.at[pl.ds((cid * NT + tid) * ORPT, ORPT)], idx_v)
    for mb in range(OMB):
        mrow0 = tid * ORPT + mb * CHUNK
        for g in range(CHUNK // 16):
            mrows_v[mb, pl.ds(g * 16, 16)] = mrow0 + g * 16 + iota16
    pltpu.sync_copy(zeros_in.at[pl.ds(0, ORPT)], acc.at[pl.ds(tid * ORPT, ORPT)])
    plsc.subcore_barrier()
    for l in range(N_LAYERS + 1):
        for mb in range(OMB):
            pltpu.async_copy(
                srcs[l].at[idx_v.at[pl.ds(mb * CHUNK, CHUNK)]], gb0, sem
            ).wait()

            def cv_body(r, _):
                v = gb0[r, pl.ds(0, 16)]
                sb0[r, pl.ds(0, 16)] = plsc.bitcast(
                    lax.shift_left(v, 16), jnp.float32)
                sb0[r, pl.ds(16, 16)] = plsc.bitcast(
                    v & jnp.int32(-65536), jnp.float32)
                return 0

            lax.fori_loop(0, CHUNK, cv_body, 0)
            pltpu.sync_copy(sb0, acc.at[mrows_v.at[mb]], add=True)
    plsc.subcore_barrier()
    for mb in range(OMB):
        pltpu.sync_copy(acc.at[pl.ds(tid * ORPT + mb * CHUNK, CHUNK)], sb0)

        def scale_body(r, _):
            for h in range(HALF // 16):
                x = sb0[r, pl.ds(h * 16, 16)]
                sb0[r, pl.ds(h * 16, 16)] = x * 0.25
            return 0

        lax.fori_loop(0, CHUNK, scale_body, 0)
        pltpu.sync_copy(sb0, out_h.at[cid, pl.ds(tid * ORPT + mb * CHUNK, CHUNK)])


@jax.jit
def _lgcn_sc(ego0, rows2, cols1, vals1, idx1, zeros_in):
    mesh = plsc.VectorSubcoreMesh(core_axis_name="c", subcore_axis_name="s")
    f32 = jnp.float32
    run = functools.partial(
        pl.kernel,
        mesh=mesh,
        compiler_params=pltpu.CompilerParams(
            use_tc_tiling_on_sc=False, needs_layout_passes=False),
        out_type=(
            jax.ShapeDtypeStruct((NC, OUT_ROWS, HALF), f32),
            jax.ShapeDtypeStruct((NC * NP, HALF // 2), jnp.int32),
            jax.ShapeDtypeStruct((NC * NP, HALF // 2), jnp.int32),
            jax.ShapeDtypeStruct((NC * NP, HALF // 2), jnp.int32),
        ),
        scratch_types=[
            pltpu.VMEM_SHARED((NP, HALF), f32),
            pltpu.VMEM((BLK, CHUNK), jnp.int32),
            pltpu.VMEM((EBLK,), jnp.int32),
            pltpu.VMEM((EBLK,), f32),
            pltpu.VMEM((CHUNK, HALF // 2), jnp.int32),
            pltpu.VMEM((CHUNK, HALF // 2), jnp.int32),
            pltpu.VMEM((CHUNK, HALF // 2), jnp.int32),
            pltpu.VMEM((CHUNK, HALF // 2), jnp.int32),
            pltpu.VMEM((CHUNK, HALF), f32),
            pltpu.VMEM((CHUNK, HALF), f32),
            pltpu.VMEM((ORPT,), jnp.int32),
            pltpu.VMEM((OMB, CHUNK), jnp.int32),
            pltpu.SemaphoreType.DMA,
            pltpu.SemaphoreType.DMA,
            pltpu.SemaphoreType.DMA,
            pltpu.SemaphoreType.DMA,
            pltpu.SemaphoreType.DMA,
            pltpu.SemaphoreType.DMA,
            pltpu.SemaphoreType.DMA,
        ],
    )(_lgcn_body)
    return run(ego0, rows2, cols1, vals1, idx1, zeros_in)


def kernel(user_emb, item_emb, adj_vals, codebook, adj_rows, adj_cols, users, items):
    ego = jnp.concatenate([user_emb, item_emb], axis=0)
    egoh = ego.reshape(N_NODES, NC, HALF).transpose(1, 0, 2)
    ego_bf = jnp.pad(egoh, ((0, 0), (0, NP - N_NODES), (0, 0))).reshape(
        NC * NP, HALF).astype(jnp.bfloat16)
    ego0 = lax.bitcast_convert_type(
        ego_bf.reshape(NC * NP, HALF // 2, 2), jnp.int32)

    pad = ((0, 0), (0, EPTP - EPT))
    rows_p = jnp.pad(adj_rows.reshape(NT, EPT), pad)
    cols_p = jnp.pad(adj_cols.reshape(NT, EPT), pad)
    vals_p = jnp.pad(adj_vals.reshape(NT, EPT), pad)
    rows2 = rows_p.reshape(NT * CPT, CHUNK)
    cols_f = cols_p.reshape(E_PAD)
    cols1 = jnp.concatenate([cols_f, cols_f + NP])
    vals1 = vals_p.reshape(E_PAD)

    all_idx = jnp.concatenate([users, items + USER_COUNT])
    idx1 = (all_idx[None, :] + jnp.array([0, NP], jnp.int32)[:, None]
            ).reshape(NC * OUT_ROWS)
    zeros_in = jnp.zeros((RPT, HALF), jnp.float32)

    out_h, _, _, _ = _lgcn_sc(ego0, rows2, cols1, vals1, idx1, zeros_in)

    perm = (jnp.arange(HALF) // 2) + 16 * (jnp.arange(HALF) % 2)
    x = out_h[:, :, perm].transpose(1, 0, 2).reshape(OUT_ROWS, EMB)
    user_embeddings = x[:BATCH]
    item_embeddings = x[BATCH:]
    return (user_embeddings, item_embeddings, jnp.zeros((), jnp.float32))


# speedup vs baseline: 1.5063x; 1.5063x over previous
import functools

import jax
import jax.numpy as jnp
from jax import lax
from jax.experimental import pallas as pl
from jax.experimental.pallas import tpu as pltpu
from jax.experimental.pallas import tpu_sc as plsc

USER_COUNT = 25000
ITEM_COUNT = 25000
N_NODES = USER_COUNT + ITEM_COUNT
N_EDGES = 800000
EMB = 64
HALF = 32
BATCH = 4096
N_LAYERS = 3

NC = 2
NT = 16

EPT = N_EDGES // NT
CHUNK = 128
CPT = 392
EPTP = CPT * CHUNK
E_PAD = NT * EPTP
BLK = 28
NBUF = 4
DEPTH = 3
NBLK = CPT // BLK
EBLK = BLK * CHUNK

NP = 50048
RPT = NP // NT
OUT_ROWS = 2 * BATCH
ORPT = OUT_ROWS // NT
OMB = ORPT // CHUNK


def _lgcn_body(ego0, rows2, cols1, vals1, idx1, zeros_in,
               out_h, e1, e2, e3,
               acc, rows_v, cols_v, vals_v, rb0, rb1, rb2, rb3,
               idx_v, mrows_v,
               gs0, gs1, gs2, gs3, ss0, ss1, ss2, ss3, sem):
    cid = lax.axis_index("c")
    tid = lax.axis_index("s")
    iota16 = lax.iota(jnp.int32, 16)

    srcs = [ego0, e1, e2, e3]

    for layer in range(N_LAYERS):
        src = srcs[layer]
        dst = srcs[layer + 1]
        pltpu.sync_copy(zeros_in, acc.at[pl.ds(tid * RPT, RPT)])
        plsc.subcore_barrier()

        bufs = (rb0, rb1, rb2, rb3)
        sems = (gs0, gs1, gs2, gs3)
        ssems = (ss0, ss1, ss2, ss3)

        def blk_body(b, _, src=src):
            cb = tid * CPT + b * BLK
            eb = cb * CHUNK
            pltpu.sync_copy(rows2.at[pl.ds(cb, BLK)], rows_v)
            pltpu.sync_copy(cols1.at[pl.ds(cid * E_PAD + eb, EBLK)], cols_v)
            pltpu.sync_copy(vals1.at[pl.ds(eb, EBLK)], vals_v)

            def fire(kq, p):
                pltpu.async_copy(
                    src.at[cols_v.at[pl.ds(kq * CHUNK, CHUNK)]],
                    bufs[p], sems[p])

            def scale(buf, k):
                def grp_body(g, _):
                    vv = vals_v[pl.ds(k * CHUNK + g * 16, 16)]
                    for j in range(16):
                        bv = vv.at[jnp.full((16,), j, jnp.int32)].get(
                            mode="promise_in_bounds")
                        r = g * 16 + j
                        for h in range(HALF // 16):
                            x = buf[r, pl.ds(h * 16, 16)]
                            buf[r, pl.ds(h * 16, 16)] = x * bv
                    return 0

                lax.fori_loop(0, CHUNK // 16, grp_body, 0)

            for p in range(DEPTH):
                fire(p, p)

            def quad_body(kk, _):
                for p in range(NBUF):
                    k = kk * NBUF + p
                    pltpu.make_async_copy(
                        zeros_in.at[pl.ds(0, CHUNK)], bufs[p], sems[p]
                    ).wait()
                    q = (p + DEPTH) % NBUF
                    if p >= NBUF - DEPTH:
                        pltpu.make_async_copy(
                            zeros_in.at[pl.ds(0, CHUNK)], bufs[q], ssems[q]
                        ).wait()
                    else:
                        @pl.when(kk > 0)
                        def _(q=q):
                            pltpu.make_async_copy(
                                zeros_in.at[pl.ds(0, CHUNK)], bufs[q], ssems[q]
                            ).wait()

                    @pl.when(k + DEPTH < BLK)
                    def _(k=k, q=q):
                        fire(k + DEPTH, q)

                    scale(bufs[p], k)
                    pltpu.async_copy(bufs[p], acc.at[rows_v.at[k]],
                                     ssems[p], add=True)
                return 0

            lax.fori_loop(0, BLK // NBUF, quad_body, 0)
            for k in range(BLK - NBUF + DEPTH, BLK):
                pltpu.make_async_copy(
                    zeros_in.at[pl.ds(0, CHUNK)], bufs[k % NBUF],
                    ssems[k % NBUF]).wait()
            return 0

        lax.fori_loop(0, NBLK, blk_body, 0)
        plsc.subcore_barrier()
        pltpu.sync_copy(
            acc.at[pl.ds(tid * RPT, RPT)],
            dst.at[pl.ds(cid * NP + tid * RPT, RPT)],
        )
        plsc.subcore_barrier()

    pltpu.sync_copy(idx1.at[pl.ds((cid * NT + tid) * ORPT, ORPT)], idx_v)
    for mb in range(OMB):
        mrow0 = tid * ORPT + mb * CHUNK
        for g in range(CHUNK // 16):
            mrows_v[mb, pl.ds(g * 16, 16)] = mrow0 + g * 16 + iota16
    pltpu.sync_copy(zeros_in.at[pl.ds(0, ORPT)], acc.at[pl.ds(tid * ORPT, ORPT)])
    plsc.subcore_barrier()
    for l in range(N_LAYERS + 1):
        for mb in range(OMB):
            pltpu.async_copy(
                srcs[l].at[idx_v.at[pl.ds(mb * CHUNK, CHUNK)]], rb0, sem
            ).wait()
            pltpu.sync_copy(rb0, acc.at[mrows_v.at[mb]], add=True)
    plsc.subcore_barrier()
    for mb in range(OMB):
        pltpu.sync_copy(acc.at[pl.ds(tid * ORPT + mb * CHUNK, CHUNK)], rb0)

        def scale_body(r, _):
            for h in range(HALF // 16):
                x = rb0[r, pl.ds(h * 16, 16)]
                rb0[r, pl.ds(h * 16, 16)] = x * 0.25
            return 0

        lax.fori_loop(0, CHUNK, scale_body, 0)
        pltpu.sync_copy(rb0, out_h.at[cid, pl.ds(tid * ORPT + mb * CHUNK, CHUNK)])


@jax.jit
def _lgcn_sc(ego0, rows2, cols1, vals1, idx1, zeros_in):
    mesh = plsc.VectorSubcoreMesh(core_axis_name="c", subcore_axis_name="s")
    f32 = jnp.float32
    run = functools.partial(
        pl.kernel,
        mesh=mesh,
        compiler_params=pltpu.CompilerParams(use_tc_tiling_on_sc=False),
        out_type=(
            jax.ShapeDtypeStruct((NC, OUT_ROWS, HALF), f32),
            jax.ShapeDtypeStruct((NC * NP, HALF), f32),
            jax.ShapeDtypeStruct((NC * NP, HALF), f32),
            jax.ShapeDtypeStruct((NC * NP, HALF), f32),
        ),
        scratch_types=[
            pltpu.VMEM_SHARED((NP, HALF), f32),
            pltpu.VMEM((BLK, CHUNK), jnp.int32),
            pltpu.VMEM((EBLK,), jnp.int32),
            pltpu.VMEM((EBLK,), f32),
            pltpu.VMEM((CHUNK, HALF), f32),
            pltpu.VMEM((CHUNK, HALF), f32),
            pltpu.VMEM((CHUNK, HALF), f32),
            pltpu.VMEM((CHUNK, HALF), f32),
            pltpu.VMEM((ORPT,), jnp.int32),
            pltpu.VMEM((OMB, CHUNK), jnp.int32),
            pltpu.SemaphoreType.DMA,
            pltpu.SemaphoreType.DMA,
            pltpu.SemaphoreType.DMA,
            pltpu.SemaphoreType.DMA,
            pltpu.SemaphoreType.DMA,
            pltpu.SemaphoreType.DMA,
            pltpu.SemaphoreType.DMA,
            pltpu.SemaphoreType.DMA,
            pltpu.SemaphoreType.DMA,
        ],
    )(_lgcn_body)
    return run(ego0, rows2, cols1, vals1, idx1, zeros_in)


def kernel(user_emb, item_emb, adj_vals, codebook, adj_rows, adj_cols, users, items):
    ego = jnp.concatenate([user_emb, item_emb], axis=0)
    egoh = ego.reshape(N_NODES, NC, HALF).transpose(1, 0, 2)
    ego0 = jnp.pad(egoh, ((0, 0), (0, NP - N_NODES), (0, 0))).reshape(
        NC * NP, HALF)

    pad = ((0, 0), (0, EPTP - EPT))
    rows_p = jnp.pad(adj_rows.reshape(NT, EPT), pad)
    cols_p = jnp.pad(adj_cols.reshape(NT, EPT), pad)
    vals_p = jnp.pad(adj_vals.reshape(NT, EPT), pad)
    rows2 = rows_p.reshape(NT * CPT, CHUNK)
    cols_f = cols_p.reshape(E_PAD)
    cols1 = jnp.concatenate([cols_f, cols_f + NP])
    vals1 = vals_p.reshape(E_PAD)

    all_idx = jnp.concatenate([users, items + USER_COUNT])
    idx1 = (all_idx[None, :] + jnp.array([0, NP], jnp.int32)[:, None]
            ).reshape(NC * OUT_ROWS)
    zeros_in = jnp.zeros((RPT, HALF), jnp.float32)

    out_h, _, _, _ = _lgcn_sc(ego0, rows2, cols1, vals1, idx1, zeros_in)

    x = out_h.transpose(1, 0, 2).reshape(OUT_ROWS, EMB)
    user_embeddings = x[:BATCH]
    item_embeddings = x[BATCH:]
    return (user_embeddings, item_embeddings, jnp.zeros((), jnp.float32))


# speedup vs baseline: 1.5433x; 1.0245x over previous
import functools

import jax
import jax.numpy as jnp
from jax import lax
from jax.experimental import pallas as pl
from jax.experimental.pallas import tpu as pltpu
from jax.experimental.pallas import tpu_sc as plsc

USER_COUNT = 25000
ITEM_COUNT = 25000
N_NODES = USER_COUNT + ITEM_COUNT
N_EDGES = 800000
EMB = 64
HALF = 32
BATCH = 4096
N_LAYERS = 3

NC = 2
NT = 16

EPT = N_EDGES // NT
CHUNK = 128
CPT = 392
EPTP = CPT * CHUNK
E_PAD = NT * EPTP
BLK = 28
NBUF = 4
DEPTH = 2
NBLK = CPT // BLK
EBLK = BLK * CHUNK

NP = 50048
RPT = NP // NT
OUT_ROWS = 2 * BATCH
ORPT = OUT_ROWS // NT
OMB = ORPT // CHUNK


def _lgcn_body(ego0, rows2, cols1, vals1, idx1, zeros_in,
               out_h, e1, e2, e3,
               acc, rows_v, cols_v, vals_v, rb0, rb1, rb2, rb3,
               idx_v, mrows_v,
               gs0, gs1, gs2, gs3, ss0, ss1, ss2, ss3, sem):
    cid = lax.axis_index("c")
    tid = lax.axis_index("s")
    iota16 = lax.iota(jnp.int32, 16)

    srcs = [ego0, e1, e2, e3]

    for layer in range(N_LAYERS):
        src = srcs[layer]
        dst = srcs[layer + 1]
        pltpu.sync_copy(zeros_in, acc.at[pl.ds(tid * RPT, RPT)])
        plsc.subcore_barrier()

        bufs = (rb0, rb1, rb2, rb3)
        sems = (gs0, gs1, gs2, gs3)
        ssems = (ss0, ss1, ss2, ss3)

        def blk_body(b, _, src=src):
            cb = tid * CPT + b * BLK
            eb = cb * CHUNK
            pltpu.sync_copy(rows2.at[pl.ds(cb, BLK)], rows_v)
            pltpu.sync_copy(cols1.at[pl.ds(cid * E_PAD + eb, EBLK)], cols_v)
            pltpu.sync_copy(vals1.at[pl.ds(eb, EBLK)], vals_v)

            def fire(kq, p):
                pltpu.async_copy(
                    src.at[cols_v.at[pl.ds(kq * CHUNK, CHUNK)]],
                    bufs[p], sems[p])

            def scale(buf, k):
                def grp_body(g, _):
                    vv = vals_v[pl.ds(k * CHUNK + g * 16, 16)]
                    for j in range(16):
                        bv = vv.at[jnp.full((16,), j, jnp.int32)].get(
                            mode="promise_in_bounds")
                        r = g * 16 + j
                        for h in range(HALF // 16):
                            x = buf[r, pl.ds(h * 16, 16)]
                            buf[r, pl.ds(h * 16, 16)] = x * bv
                    return 0

                lax.fori_loop(0, CHUNK // 16, grp_body, 0)

            for p in range(DEPTH):
                fire(p, p)

            def quad_body(kk, _):
                for p in range(NBUF):
                    k = kk * NBUF + p
                    pltpu.make_async_copy(
                        zeros_in.at[pl.ds(0, CHUNK)], bufs[p], sems[p]
                    ).wait()
                    q = (p + DEPTH) % NBUF
                    if p >= NBUF - DEPTH:
                        pltpu.make_async_copy(
                            zeros_in.at[pl.ds(0, CHUNK)], bufs[q], ssems[q]
                        ).wait()
                    else:
                        @pl.when(kk > 0)
                        def _(q=q):
                            pltpu.make_async_copy(
                                zeros_in.at[pl.ds(0, CHUNK)], bufs[q], ssems[q]
                            ).wait()

                    @pl.when(k + DEPTH < BLK)
                    def _(k=k, q=q):
                        fire(k + DEPTH, q)

                    scale(bufs[p], k)
                    pltpu.async_copy(bufs[p], acc.at[rows_v.at[k]],
                                     ssems[p], add=True)
                return 0

            lax.fori_loop(0, BLK // NBUF, quad_body, 0)
            for k in range(BLK - NBUF + DEPTH, BLK):
                pltpu.make_async_copy(
                    zeros_in.at[pl.ds(0, CHUNK)], bufs[k % NBUF],
                    ssems[k % NBUF]).wait()
            return 0

        lax.fori_loop(0, NBLK, blk_body, 0)
        plsc.subcore_barrier()
        pltpu.sync_copy(
            acc.at[pl.ds(tid * RPT, RPT)],
            dst.at[pl.ds(cid * NP + tid * RPT, RPT)],
        )
        plsc.subcore_barrier()

    pltpu.sync_copy(idx1.at[pl.ds((cid * NT + tid) * ORPT, ORPT)], idx_v)
    for mb in range(OMB):
        mrow0 = tid * ORPT + mb * CHUNK
        for g in range(CHUNK // 16):
            mrows_v[mb, pl.ds(g * 16, 16)] = mrow0 + g * 16 + iota16
    pltpu.sync_copy(zeros_in.at[pl.ds(0, ORPT)], acc.at[pl.ds(tid * ORPT, ORPT)])
    plsc.subcore_barrier()
    for l in range(N_LAYERS + 1):
        for mb in range(OMB):
            pltpu.async_copy(
                srcs[l].at[idx_v.at[pl.ds(mb * CHUNK, CHUNK)]], rb0, sem
            ).wait()
            pltpu.sync_copy(rb0, acc.at[mrows_v.at[mb]], add=True)
    plsc.subcore_barrier()
    for mb in range(OMB):
        pltpu.sync_copy(acc.at[pl.ds(tid * ORPT + mb * CHUNK, CHUNK)], rb0)

        def scale_body(r, _):
            for h in range(HALF // 16):
                x = rb0[r, pl.ds(h * 16, 16)]
                rb0[r, pl.ds(h * 16, 16)] = x * 0.25
            return 0

        lax.fori_loop(0, CHUNK, scale_body, 0)
        pltpu.sync_copy(rb0, out_h.at[cid, pl.ds(tid * ORPT + mb * CHUNK, CHUNK)])


@jax.jit
def _lgcn_sc(ego0, rows2, cols1, vals1, idx1, zeros_in):
    mesh = plsc.VectorSubcoreMesh(core_axis_name="c", subcore_axis_name="s")
    f32 = jnp.float32
    run = functools.partial(
        pl.kernel,
        mesh=mesh,
        compiler_params=pltpu.CompilerParams(use_tc_tiling_on_sc=False),
        out_type=(
            jax.ShapeDtypeStruct((NC, OUT_ROWS, HALF), f32),
            jax.ShapeDtypeStruct((NC * NP, HALF), f32),
            jax.ShapeDtypeStruct((NC * NP, HALF), f32),
            jax.ShapeDtypeStruct((NC * NP, HALF), f32),
        ),
        scratch_types=[
            pltpu.VMEM_SHARED((NP, HALF), f32),
            pltpu.VMEM((BLK, CHUNK), jnp.int32),
            pltpu.VMEM((EBLK,), jnp.int32),
            pltpu.VMEM((EBLK,), f32),
            pltpu.VMEM((CHUNK, HALF), f32),
            pltpu.VMEM((CHUNK, HALF), f32),
            pltpu.VMEM((CHUNK, HALF), f32),
            pltpu.VMEM((CHUNK, HALF), f32),
            pltpu.VMEM((ORPT,), jnp.int32),
            pltpu.VMEM((OMB, CHUNK), jnp.int32),
            pltpu.SemaphoreType.DMA,
            pltpu.SemaphoreType.DMA,
            pltpu.SemaphoreType.DMA,
            pltpu.SemaphoreType.DMA,
            pltpu.SemaphoreType.DMA,
            pltpu.SemaphoreType.DMA,
            pltpu.SemaphoreType.DMA,
            pltpu.SemaphoreType.DMA,
            pltpu.SemaphoreType.DMA,
        ],
    )(_lgcn_body)
    return run(ego0, rows2, cols1, vals1, idx1, zeros_in)


def kernel(user_emb, item_emb, adj_vals, codebook, adj_rows, adj_cols, users, items):
    ego = jnp.concatenate([user_emb, item_emb], axis=0)
    egoh = ego.reshape(N_NODES, NC, HALF).transpose(1, 0, 2)
    ego0 = jnp.pad(egoh, ((0, 0), (0, NP - N_NODES), (0, 0))).reshape(
        NC * NP, HALF)

    pad = ((0, 0), (0, EPTP - EPT))
    rows_p = jnp.pad(adj_rows.reshape(NT, EPT), pad)
    cols_p = jnp.pad(adj_cols.reshape(NT, EPT), pad)
    vals_p = jnp.pad(adj_vals.reshape(NT, EPT), pad)
    rows2 = rows_p.reshape(NT * CPT, CHUNK)
    cols_f = cols_p.reshape(E_PAD)
    cols1 = jnp.concatenate([cols_f, cols_f + NP])
    vals1 = vals_p.reshape(E_PAD)

    all_idx = jnp.concatenate([users, items + USER_COUNT])
    idx1 = (all_idx[None, :] + jnp.array([0, NP], jnp.int32)[:, None]
            ).reshape(NC * OUT_ROWS)
    zeros_in = jnp.zeros((RPT, HALF), jnp.float32)

    out_h, _, _, _ = _lgcn_sc(ego0, rows2, cols1, vals1, idx1, zeros_in)

    x = out_h.transpose(1, 0, 2).reshape(OUT_ROWS, EMB)
    user_embeddings = x[:BATCH]
    item_embeddings = x[BATCH:]
    return (user_embeddings, item_embeddings, jnp.zeros((), jnp.float32))
